# baseline (speedup 1.0000x reference)
import jax
import jax.numpy as jnp
from jax.experimental import pallas as pl
from jax.experimental.pallas import tpu as pltpu

_S = 64.0
_M = 0.4

_RB = 8
_CB = 8192


def _scale_fix_kernel(lab_ref, neg_ref, cos_ref, out_ref):
    out_ref[...] = cos_ref[...] * _S
    row0 = pl.program_id(0) * _RB
    col0 = pl.program_id(1) * _CB
    for r in range(_RB):
        off = lab_ref[row0 + r] - col0
        @pl.when(jnp.logical_and(off >= 0, off < _CB))
        def _fix(r=r, off=off):
            base = pl.multiple_of((off // 128) * 128, 128)
            lane = off - base
            tile = cos_ref[pl.ds(r, 1), pl.ds(base, 128)] * _S
            io = jax.lax.broadcasted_iota(jnp.int32, (1, 128), 1)
            out_ref[pl.ds(r, 1), pl.ds(base, 128)] = tile + jnp.where(
                io == lane, neg_ref[row0 + r], 0.0
            )


def kernel(cosine, label, qs_scores):
    del qs_scores
    B, C = cosine.shape
    mkey = jax.random.fold_in(jax.random.key(0), 123)
    margin = _M + 0.05 * jax.random.normal(mkey, (B,), dtype=jnp.float32)
    neg = -_S * margin

    grid = (B // _RB, pl.cdiv(C, _CB))
    return pl.pallas_call(
        _scale_fix_kernel,
        grid=grid,
        in_specs=[
            pl.BlockSpec((B,), lambda i, j: (0,), memory_space=pltpu.SMEM),
            pl.BlockSpec((B,), lambda i, j: (0,), memory_space=pltpu.SMEM),
            pl.BlockSpec((_RB, _CB), lambda i, j: (i, j)),
        ],
        out_specs=pl.BlockSpec((_RB, _CB), lambda i, j: (i, j)),
        out_shape=jax.ShapeDtypeStruct((B, C), cosine.dtype),
    )(label, neg, cosine)


# speedup vs baseline: 1.6708x; 1.6708x over previous
import jax
import jax.numpy as jnp
from jax.experimental import pallas as pl
from jax.experimental.pallas import tpu as pltpu

_S = 64.0
_M = 0.4

_RB = 64
_CB = 8192


def _scale_fix_kernel(lab_ref, neg_ref, cos_ref, out_ref):
    out_ref[...] = cos_ref[...] * _S
    row0 = pl.program_id(0) * _RB
    col0 = pl.program_id(1) * _CB
    for r in range(_RB):
        off = lab_ref[row0 + r] - col0
        @pl.when(jnp.logical_and(off >= 0, off < _CB))
        def _fix(r=r, off=off):
            base = pl.multiple_of((off // 128) * 128, 128)
            lane = off - base
            tile = cos_ref[pl.ds(r, 1), pl.ds(base, 128)] * _S
            io = jax.lax.broadcasted_iota(jnp.int32, (1, 128), 1)
            out_ref[pl.ds(r, 1), pl.ds(base, 128)] = tile + jnp.where(
                io == lane, neg_ref[row0 + r], 0.0
            )


def kernel(cosine, label, qs_scores):
    del qs_scores
    B, C = cosine.shape
    mkey = jax.random.fold_in(jax.random.key(0), 123)
    margin = _M + 0.05 * jax.random.normal(mkey, (B,), dtype=jnp.float32)
    neg = -_S * margin

    grid = (B // _RB, pl.cdiv(C, _CB))
    return pl.pallas_call(
        _scale_fix_kernel,
        grid=grid,
        in_specs=[
            pl.BlockSpec((B,), lambda i, j: (0,), memory_space=pltpu.SMEM),
            pl.BlockSpec((B,), lambda i, j: (0,), memory_space=pltpu.SMEM),
            pl.BlockSpec((_RB, _CB), lambda i, j: (i, j)),
        ],
        out_specs=pl.BlockSpec((_RB, _CB), lambda i, j: (i, j)),
        out_shape=jax.ShapeDtypeStruct((B, C), cosine.dtype),
    )(label, neg, cosine)


# speedup vs baseline: 1.6721x; 1.0008x over previous
import jax
import jax.numpy as jnp
from jax.experimental import pallas as pl
from jax.experimental.pallas import tpu as pltpu

_S = 64.0
_M = 0.4

_RB = 64
_CB = 8192


def _scale_fix_kernel(lab_ref, neg_ref, cos_ref, out_ref):
    out_ref[...] = cos_ref[...] * _S
    row0 = pl.program_id(0) * _RB
    col0 = pl.program_id(1) * _CB
    for r in range(_RB):
        off = lab_ref[row0 + r] - col0
        @pl.when(jnp.logical_and(off >= 0, off < _CB))
        def _fix(r=r, off=off):
            base = pl.multiple_of((off // 128) * 128, 128)
            lane = off - base
            tile = cos_ref[pl.ds(r, 1), pl.ds(base, 128)] * _S
            io = jax.lax.broadcasted_iota(jnp.int32, (1, 128), 1)
            out_ref[pl.ds(r, 1), pl.ds(base, 128)] = tile + jnp.where(
                io == lane, neg_ref[row0 + r], 0.0
            )


def kernel(cosine, label, qs_scores):
    del qs_scores
    B, C = cosine.shape
    mkey = jax.random.fold_in(jax.random.key(0), 123)
    margin = _M + 0.05 * jax.random.normal(mkey, (B,), dtype=jnp.float32)
    neg = -_S * margin

    grid = (B // _RB, pl.cdiv(C, _CB))
    return pl.pallas_call(
        _scale_fix_kernel,
        grid=grid,
        in_specs=[
            pl.BlockSpec((B,), lambda i, j: (0,), memory_space=pltpu.SMEM),
            pl.BlockSpec((B,), lambda i, j: (0,), memory_space=pltpu.SMEM),
            pl.BlockSpec((_RB, _CB), lambda i, j: (i, j)),
        ],
        out_specs=pl.BlockSpec((_RB, _CB), lambda i, j: (i, j)),
        out_shape=jax.ShapeDtypeStruct((B, C), cosine.dtype),
        compiler_params=pltpu.CompilerParams(
            dimension_semantics=("parallel", "parallel"),
        ),
    )(label, neg, cosine)


# speedup vs baseline: 1.6934x; 1.0128x over previous
import jax
import jax.numpy as jnp
from jax.experimental import pallas as pl
from jax.experimental.pallas import tpu as pltpu

_S = 64.0
_M = 0.4

_RB = 256
_CB = 2048


def _make_body(ncol_blocks):
    def _body(starts_ref, hrow_ref, hlab_ref, hneg_ref, cos_ref, out_ref):
        out_ref[...] = cos_ref[...] * _S
        i = pl.program_id(0)
        j = pl.program_id(1)
        cell = i * ncol_blocks + j
        s0 = starts_ref[cell]
        s1 = starts_ref[cell + 1]

        def _fix(k, carry):
            r = hrow_ref[k] - i * _RB
            off = hlab_ref[k] - j * _CB
            br = pl.multiple_of((r // 8) * 8, 8)
            bc = pl.multiple_of((off // 128) * 128, 128)
            io_r = jax.lax.broadcasted_iota(jnp.int32, (8, 128), 0)
            io_c = jax.lax.broadcasted_iota(jnp.int32, (8, 128), 1)
            sel = jnp.logical_and(io_r == r - br, io_c == off - bc)
            tile = out_ref[pl.ds(br, 8), pl.ds(bc, 128)]
            out_ref[pl.ds(br, 8), pl.ds(bc, 128)] = tile + jnp.where(
                sel, hneg_ref[k], 0.0
            )
            return carry

        jax.lax.fori_loop(s0, s1, _fix, 0)

    return _body


def kernel(cosine, label, qs_scores):
    del qs_scores
    B, C = cosine.shape
    mkey = jax.random.fold_in(jax.random.key(0), 123)
    margin = _M + 0.05 * jax.random.normal(mkey, (B,), dtype=jnp.float32)
    neg = -_S * margin

    nrow = B // _RB
    ncol = pl.cdiv(C, _CB)
    ncells = nrow * ncol
    cell = (jnp.arange(B, dtype=jnp.int32) // _RB) * ncol + label // _CB
    order = jnp.argsort(cell).astype(jnp.int32)
    starts = jnp.searchsorted(
        cell[order], jnp.arange(ncells + 1, dtype=jnp.int32)
    ).astype(jnp.int32)

    return pl.pallas_call(
        _make_body(ncol),
        grid=(nrow, ncol),
        in_specs=[
            pl.BlockSpec(memory_space=pltpu.SMEM),
            pl.BlockSpec(memory_space=pltpu.SMEM),
            pl.BlockSpec(memory_space=pltpu.SMEM),
            pl.BlockSpec(memory_space=pltpu.SMEM),
            pl.BlockSpec((_RB, _CB), lambda i, j: (i, j)),
        ],
        out_specs=pl.BlockSpec((_RB, _CB), lambda i, j: (i, j)),
        out_shape=jax.ShapeDtypeStruct((B, C), cosine.dtype),
    )(starts, order, label[order], neg[order], cosine)


# speedup vs baseline: 1.7476x; 1.0320x over previous
import jax
import jax.numpy as jnp
from jax.experimental import pallas as pl
from jax.experimental.pallas import tpu as pltpu

_S = 64.0
_M = 0.4

_RB = 512
_CB = 2048


def _make_body(ncol_blocks):
    def _body(starts_ref, hrow_ref, hlab_ref, hneg_ref, cos_ref, out_ref):
        out_ref[...] = cos_ref[...] * _S
        i = pl.program_id(0)
        j = pl.program_id(1)
        cell = i * ncol_blocks + j
        s0 = starts_ref[cell]
        s1 = starts_ref[cell + 1]

        def _fix(k, carry):
            r = hrow_ref[k] - i * _RB
            off = hlab_ref[k] - j * _CB
            br = pl.multiple_of((r // 8) * 8, 8)
            bc = pl.multiple_of((off // 128) * 128, 128)
            io_r = jax.lax.broadcasted_iota(jnp.int32, (8, 128), 0)
            io_c = jax.lax.broadcasted_iota(jnp.int32, (8, 128), 1)
            sel = jnp.logical_and(io_r == r - br, io_c == off - bc)
            tile = out_ref[pl.ds(br, 8), pl.ds(bc, 128)]
            out_ref[pl.ds(br, 8), pl.ds(bc, 128)] = tile + jnp.where(
                sel, hneg_ref[k], 0.0
            )
            return carry

        jax.lax.fori_loop(s0, s1, _fix, 0)

    return _body


def kernel(cosine, label, qs_scores):
    del qs_scores
    B, C = cosine.shape
    mkey = jax.random.fold_in(jax.random.key(0), 123)
    margin = _M + 0.05 * jax.random.normal(mkey, (B,), dtype=jnp.float32)
    neg = -_S * margin

    nrow = B // _RB
    ncol = pl.cdiv(C, _CB)
    ncells = nrow * ncol
    cell = (jnp.arange(B, dtype=jnp.int32) // _RB) * ncol + label // _CB
    order = jnp.argsort(cell).astype(jnp.int32)
    starts = jnp.searchsorted(
        cell[order], jnp.arange(ncells + 1, dtype=jnp.int32)
    ).astype(jnp.int32)

    return pl.pallas_call(
        _make_body(ncol),
        grid=(nrow, ncol),
        in_specs=[
            pl.BlockSpec(memory_space=pltpu.SMEM),
            pl.BlockSpec(memory_space=pltpu.SMEM),
            pl.BlockSpec(memory_space=pltpu.SMEM),
            pl.BlockSpec(memory_space=pltpu.SMEM),
            pl.BlockSpec((_RB, _CB), lambda i, j: (i, j)),
        ],
        out_specs=pl.BlockSpec((_RB, _CB), lambda i, j: (i, j)),
        out_shape=jax.ShapeDtypeStruct((B, C), cosine.dtype),
    )(starts, order, label[order], neg[order], cosine)


# speedup vs baseline: 1.7588x; 1.0064x over previous
import jax
import jax.numpy as jnp
from jax.experimental import pallas as pl
from jax.experimental.pallas import tpu as pltpu

_S = 64.0
_M = 0.4

_RB = 1024
_CB = 2048


def _make_body(ncol_blocks):
    def _body(starts_ref, hrow_ref, hlab_ref, hneg_ref, cos_ref, out_ref):
        out_ref[...] = cos_ref[...] * _S
        i = pl.program_id(0)
        j = pl.program_id(1)
        cell = i * ncol_blocks + j
        s0 = starts_ref[cell]
        s1 = starts_ref[cell + 1]

        def _fix(k, carry):
            r = hrow_ref[k] - i * _RB
            off = hlab_ref[k] - j * _CB
            br = pl.multiple_of((r // 8) * 8, 8)
            bc = pl.multiple_of((off // 128) * 128, 128)
            io_r = jax.lax.broadcasted_iota(jnp.int32, (8, 128), 0)
            io_c = jax.lax.broadcasted_iota(jnp.int32, (8, 128), 1)
            sel = jnp.logical_and(io_r == r - br, io_c == off - bc)
            tile = out_ref[pl.ds(br, 8), pl.ds(bc, 128)]
            out_ref[pl.ds(br, 8), pl.ds(bc, 128)] = tile + jnp.where(
                sel, hneg_ref[k], 0.0
            )
            return carry

        jax.lax.fori_loop(s0, s1, _fix, 0)

    return _body


def kernel(cosine, label, qs_scores):
    del qs_scores
    B, C = cosine.shape
    mkey = jax.random.fold_in(jax.random.key(0), 123)
    margin = _M + 0.05 * jax.random.normal(mkey, (B,), dtype=jnp.float32)
    neg = -_S * margin

    nrow = B // _RB
    ncol = pl.cdiv(C, _CB)
    ncells = nrow * ncol
    cell = (jnp.arange(B, dtype=jnp.int32) // _RB) * ncol + label // _CB
    order = jnp.argsort(cell).astype(jnp.int32)
    starts = jnp.searchsorted(
        cell[order], jnp.arange(ncells + 1, dtype=jnp.int32)
    ).astype(jnp.int32)

    return pl.pallas_call(
        _make_body(ncol),
        grid=(nrow, ncol),
        in_specs=[
            pl.BlockSpec(memory_space=pltpu.SMEM),
            pl.BlockSpec(memory_space=pltpu.SMEM),
            pl.BlockSpec(memory_space=pltpu.SMEM),
            pl.BlockSpec(memory_space=pltpu.SMEM),
            pl.BlockSpec((_RB, _CB), lambda i, j: (i, j)),
        ],
        out_specs=pl.BlockSpec((_RB, _CB), lambda i, j: (i, j)),
        out_shape=jax.ShapeDtypeStruct((B, C), cosine.dtype),
    )(starts, order, label[order], neg[order], cosine)
